# baseline (speedup 1.0000x reference)
import functools

import jax
import jax.numpy as jnp
from jax import lax
from jax.experimental import pallas as pl
from jax.experimental.pallas import tpu as pltpu
from jax.experimental.pallas import tpu_sc as plsc

N_NODES = 10000
N_EDGES = 160000
F_IN = 256
HID = 256
HH = 128
ALPHA = 0.2

NS = 16
EPT = N_EDGES // NS
CHUNK = 40
NCHUNK = EPT // CHUNK
SUP = 400
NSUP = EPT // SUP
NPAD = 10240
NT = NPAD // NS
APAD = 10048
AT = APAD // NS
OUT_PT = 624


def _dense_body(x_ref, w_ref, a2_ref, wh0_ref, wh1_ref, s12_ref):
    xw = jnp.dot(x_ref[...], w_ref[...], preferred_element_type=jnp.float32)
    wh0_ref[...] = xw[:, :HH].astype(jnp.bfloat16)
    wh1_ref[...] = xw[:, HH:].astype(jnp.bfloat16)
    s12_ref[...] = jnp.dot(xw, a2_ref[...], preferred_element_type=jnp.float32)


def _dense(x, W, a2):
    B = 1000
    grid = N_NODES // B
    return pl.pallas_call(
        _dense_body,
        grid=(grid,),
        in_specs=[
            pl.BlockSpec((B, F_IN), lambda i: (i, 0)),
            pl.BlockSpec((F_IN, HID), lambda i: (0, 0)),
            pl.BlockSpec((HID, 2), lambda i: (0, 0)),
        ],
        out_specs=[
            pl.BlockSpec((B, HH), lambda i: (i, 0)),
            pl.BlockSpec((B, HH), lambda i: (i, 0)),
            pl.BlockSpec((B, 2), lambda i: (i, 0)),
        ],
        out_shape=[
            jax.ShapeDtypeStruct((N_NODES, HH), jnp.bfloat16),
            jax.ShapeDtypeStruct((N_NODES, HH), jnp.bfloat16),
            jax.ShapeDtypeStruct((N_NODES, 2), jnp.float32),
        ],
    )(x, W, a2)


def _sc_body(wh0_h, wh1_h, s1_h, s2_h, edge_h, out_h,
             src_v, dst_v, att_v, g1a_v, g1b_v, g2a_v, g2b_v,
             rows0_v, rows1_v, rb0_v, rb1_v, zb_v,
             gsem0, gsem1, gsem2, ssem0, ssem1, ssem2,
             den_s, acc_s, s1_s, s2_s):
    c = lax.axis_index("c")
    s = lax.axis_index("s")
    zero16 = jnp.zeros((16,), jnp.float32)

    pltpu.sync_copy(edge_h.at[0, s], src_v)
    pltpu.sync_copy(edge_h.at[1, s], dst_v)

    def _zb(j, _):
        zb_v[pl.ds(j * 16, 16)] = zero16
        return 0
    lax.fori_loop(0, NT // 16, _zb, 0)
    pltpu.sync_copy(zb_v, den_s.at[pl.ds(s * NT, NT)])

    @pl.when(s == 0)
    def _():
        pltpu.sync_copy(s1_h, s1_s)
        pltpu.sync_copy(s2_h, s2_s)

    def _zrows(e, _):
        for q in range(HH // 16):
            rows0_v[e, pl.ds(q * 16, 16)] = zero16
        return 0
    lax.fori_loop(0, CHUNK, _zrows, 0)
    for q in range(AT // CHUNK):
        pltpu.sync_copy(rows0_v, acc_s.at[pl.ds(s * AT + q * CHUNK, CHUNK), :])
    pltpu.sync_copy(rows0_v.at[pl.ds(0, AT % CHUNK), :],
                    acc_s.at[pl.ds(s * AT + (AT // CHUNK) * CHUNK,
                                   AT % CHUNK), :])
    plsc.subcore_barrier()

    G1 = (g1a_v, g1b_v)
    G2 = (g2a_v, g2b_v)
    GS1 = (gsem0, gsem1)
    GS2 = (gsem2, ssem0)

    def p1_start(q, x):
        sup = pl.ds(q * SUP, SUP)
        pltpu.async_copy(s1_s.at[src_v.at[sup]], G1[x], GS1[x])
        pltpu.async_copy(s2_s.at[dst_v.at[sup]], G2[x], GS2[x])

    def p1_wait(q, x):
        sup = pl.ds(q * SUP, SUP)
        pltpu.make_async_copy(s1_s.at[src_v.at[sup]], G1[x], GS1[x]).wait()
        pltpu.make_async_copy(s2_s.at[dst_v.at[sup]], G2[x], GS2[x]).wait()

    def p1_compute(q, x):
        def _ee(r, _):
            sl = pl.ds(r * 16, 16)
            e = G1[x][sl] + G2[x][sl]
            e = jnp.where(e > 0, e, e * ALPHA)
            att_v[pl.ds(q * SUP + r * 16, 16)] = jnp.exp(e)
            return 0
        lax.fori_loop(0, SUP // 16, _ee, 0)
        sup = pl.ds(q * SUP, SUP)
        pltpu.async_copy(att_v.at[sup], den_s.at[src_v.at[sup]], ssem1,
                         add=True)

    p1_start(0, 0)

    def _p1pair(i, _):
        q0 = i * 2
        q1 = q0 + 1
        p1_wait(q0, 0)
        p1_start(q1, 1)
        p1_compute(q0, 0)
        p1_wait(q1, 1)

        @pl.when(q1 + 1 < NSUP)
        def _():
            p1_start(q1 + 1, 0)
        p1_compute(q1, 1)
        return 0
    lax.fori_loop(0, NSUP // 2, _p1pair, 0)
    p1_wait(NSUP - 1, 0)
    p1_compute(NSUP - 1, 0)

    def _dr(q, _):
        sup = pl.ds(q * SUP, SUP)
        pltpu.make_async_copy(att_v.at[sup], den_s.at[src_v.at[sup]],
                              ssem1).wait()
        return 0
    lax.fori_loop(0, NSUP, _dr, 0)
    plsc.subcore_barrier()

    GBUF = (rb0_v, rb1_v)
    FBUF = (rows0_v, rows1_v)
    GS = (gsem0, gsem1)
    SS = (ssem0, ssem1)
    MHI = jnp.int32(-65536)

    def _row_phase(wh_h, col0):
        def g_start(k, t):
            pltpu.async_copy(wh_h.at[dst_v.at[pl.ds(k * CHUNK, CHUNK)]],
                             GBUF[t], GS[t])

        def g_wait(k, t):
            pltpu.make_async_copy(
                wh_h.at[dst_v.at[pl.ds(k * CHUNK, CHUNK)]],
                GBUF[t], GS[t]).wait()

        def s_start(k, t):
            pltpu.async_copy(FBUF[t],
                             acc_s.at[src_v.at[pl.ds(k * CHUNK, CHUNK)]],
                             SS[t], add=True)

        def s_wait(k, t):
            pltpu.make_async_copy(
                FBUF[t], acc_s.at[src_v.at[pl.ds(k * CHUNK, CHUNK)]],
                SS[t]).wait()

        def _one_edge(rb, fv, e, a_s):
            for q in range(HH // 32):
                xi = rb[e, pl.ds(q * 16, 16)]
                ev = plsc.bitcast(xi << 16, jnp.float32)
                od = plsc.bitcast(xi & MHI, jnp.float32)
                fv[e, pl.ds(q * 32, 16)] = ev * a_s
                fv[e, pl.ds(q * 32 + 16, 16)] = od * a_s

        def _scale(k, t):
            rb = GBUF[t]
            fv = FBUF[t]

            def _grp(g, _):
                av = att_v[pl.ds(k * CHUNK + g * 16, 16)]
                for j in range(16):
                    _one_edge(rb, fv, g * 16 + j, av[j])
                return 0
            lax.fori_loop(0, CHUNK // 16, _grp, 0)
            av = att_v[pl.ds(k * CHUNK + (CHUNK // 16) * 16, 16)]
            for j in range(CHUNK - (CHUNK // 16) * 16):
                _one_edge(rb, fv, (CHUNK // 16) * 16 + j, av[j])

        def _step(k, t):
            g_wait(k, t)

            @pl.when(k >= 2)
            def _():
                s_wait(k - 2, t)
            _scale(k, t)

            @pl.when(k + 2 < NCHUNK)
            def _():
                g_start(k + 2, t)
            s_start(k, t)

        g_start(0, 0)
        g_start(1, 1)

        def _pair(i, _):
            k = i * 2
            _step(k, 0)
            _step(k + 1, 1)
            return 0
        lax.fori_loop(0, NCHUNK // 2, _pair, 0)
        s_wait(NCHUNK - 2, 0)
        s_wait(NCHUNK - 1, 1)
        plsc.subcore_barrier()

        def _out_block(base, nrows):
            pltpu.sync_copy(den_s.at[pl.ds(base, nrows)],
                            zb_v.at[pl.ds(0, nrows)])
            def _inv(j, _):
                sl = pl.ds(j * 16, 16)
                d = zb_v[sl]
                zb_v[sl] = jnp.where(d > 0, 1.0 / d, 0.0)
                return 0
            lax.fori_loop(0, nrows // 16, _inv, 0)

            iot2 = lax.iota(jnp.int32, 16) * 2

            def _blk(r, _):
                rsl = pl.ds(base + r * 16, 16)
                pltpu.sync_copy(acc_s.at[rsl, :], rows0_v.at[pl.ds(0, 16), :])
                av = zb_v[pl.ds(r * 16, 16)]
                for j in range(16):
                    a_s = av[j]
                    jj = jnp.full((16,), j, jnp.int32)
                    for q in range(HH // 32):
                        ev = rows0_v[j, pl.ds(q * 32, 16)] * a_s
                        od = rows0_v[j, pl.ds(q * 32 + 16, 16)] * a_s
                        plsc.store_scatter(rows1_v, [jj, iot2 + (q * 32)], ev)
                        plsc.store_scatter(rows1_v,
                                           [jj, iot2 + (q * 32 + 1)], od)
                pltpu.sync_copy(rows1_v.at[pl.ds(0, 16), :],
                                out_h.at[rsl, pl.ds(col0, HH)])
                return 0
            lax.fori_loop(0, nrows // 16, _blk, 0)

        _out_block(s * OUT_PT, OUT_PT)

        @pl.when(s == NS - 1)
        def _():
            _out_block(NS * OUT_PT, N_NODES - NS * OUT_PT)

    @pl.when(c == 0)
    def _():
        _row_phase(wh0_h, 0)

    @pl.when(c == 1)
    def _():
        _row_phase(wh1_h, HH)


_sc_attn = functools.partial(
    pl.kernel,
    out_type=jax.ShapeDtypeStruct((N_NODES, HID), jnp.float32),
    mesh=plsc.VectorSubcoreMesh(core_axis_name="c", subcore_axis_name="s"),
    compiler_params=pltpu.CompilerParams(needs_layout_passes=False,
                                         use_tc_tiling_on_sc=False),
    scratch_types=[
        pltpu.VMEM((EPT,), jnp.int32),
        pltpu.VMEM((EPT,), jnp.int32),
        pltpu.VMEM((EPT + 16,), jnp.float32),
        pltpu.VMEM((SUP,), jnp.float32),
        pltpu.VMEM((SUP,), jnp.float32),
        pltpu.VMEM((SUP,), jnp.float32),
        pltpu.VMEM((SUP,), jnp.float32),
        pltpu.VMEM((CHUNK, HH), jnp.float32),
        pltpu.VMEM((CHUNK, HH), jnp.float32),
        pltpu.VMEM((CHUNK, HH // 2), jnp.int32),
        pltpu.VMEM((CHUNK, HH // 2), jnp.int32),
        pltpu.VMEM((NT,), jnp.float32),
        pltpu.SemaphoreType.DMA,
        pltpu.SemaphoreType.DMA,
        pltpu.SemaphoreType.DMA,
        pltpu.SemaphoreType.DMA,
        pltpu.SemaphoreType.DMA,
        pltpu.SemaphoreType.DMA,
        pltpu.VMEM_SHARED((NPAD,), jnp.float32),
        pltpu.VMEM_SHARED((APAD, HH), jnp.float32),
        pltpu.VMEM_SHARED((N_NODES,), jnp.float32),
        pltpu.VMEM_SHARED((N_NODES,), jnp.float32),
    ],
)(_sc_body)


def kernel(x, edge_index, W, a):
    a2 = a.reshape(2, HID).T
    wh0, wh1, s12 = _dense(x, W, a2)
    wh0_i = lax.bitcast_convert_type(
        wh0.reshape(N_NODES, HH // 2, 2), jnp.int32)
    wh1_i = lax.bitcast_convert_type(
        wh1.reshape(N_NODES, HH // 2, 2), jnp.int32)
    edge3 = edge_index.reshape(2, NS, EPT)
    return _sc_attn(wh0_i, wh1_i, s12[:, 0], s12[:, 1], edge3)


# speedup vs baseline: 1.6547x; 1.6547x over previous
import functools

import jax
import jax.numpy as jnp
from jax import lax
from jax.experimental import pallas as pl
from jax.experimental.pallas import tpu as pltpu
from jax.experimental.pallas import tpu_sc as plsc

N_NODES = 10000
N_EDGES = 160000
F_IN = 256
HID = 256
HH = 128
ALPHA = 0.2

NS = 16
EPT = N_EDGES // NS
CHUNK = 40
NCHUNK = EPT // CHUNK
SUP = 400
NSUP = EPT // SUP
NPAD = 10240
NT = NPAD // NS
APAD = 10048
AT = APAD // NS
OUT_PT = 624


def _dense_body(x_ref, w_ref, a2_ref, wh0_ref, wh1_ref, s12_ref):
    xw = jnp.dot(x_ref[...], w_ref[...], preferred_element_type=jnp.float32)
    wh0_ref[...] = xw[:, :HH]
    wh1_ref[...] = xw[:, HH:]
    s12_ref[...] = jnp.dot(xw, a2_ref[...], preferred_element_type=jnp.float32)


def _dense(x, W, a2):
    B = 1000
    grid = N_NODES // B
    return pl.pallas_call(
        _dense_body,
        grid=(grid,),
        in_specs=[
            pl.BlockSpec((B, F_IN), lambda i: (i, 0)),
            pl.BlockSpec((F_IN, HID), lambda i: (0, 0)),
            pl.BlockSpec((HID, 2), lambda i: (0, 0)),
        ],
        out_specs=[
            pl.BlockSpec((B, HH), lambda i: (i, 0)),
            pl.BlockSpec((B, HH), lambda i: (i, 0)),
            pl.BlockSpec((B, 2), lambda i: (i, 0)),
        ],
        out_shape=[
            jax.ShapeDtypeStruct((N_NODES, HH), jnp.float32),
            jax.ShapeDtypeStruct((N_NODES, HH), jnp.float32),
            jax.ShapeDtypeStruct((N_NODES, 2), jnp.float32),
        ],
    )(x, W, a2)


def _sc_body(wh0_h, wh1_h, s1_h, s2_h, edge_h, out_h,
             src_v, dst_v, att_v, g1a_v, g1b_v, g2a_v, g2b_v,
             rows0_v, rows1_v, rows2_v, zb_v,
             gsem0, gsem1, gsem2, ssem0, ssem1, ssem2,
             den_s, acc_s, s1_s, s2_s):
    c = lax.axis_index("c")
    s = lax.axis_index("s")
    zero16 = jnp.zeros((16,), jnp.float32)

    pltpu.sync_copy(edge_h.at[0, s], src_v)
    pltpu.sync_copy(edge_h.at[1, s], dst_v)

    def _zb(j, _):
        zb_v[pl.ds(j * 16, 16)] = zero16
        return 0
    lax.fori_loop(0, NT // 16, _zb, 0)
    pltpu.sync_copy(zb_v, den_s.at[pl.ds(s * NT, NT)])

    @pl.when(s == 0)
    def _():
        pltpu.sync_copy(s1_h, s1_s)
        pltpu.sync_copy(s2_h, s2_s)

    def _zrows(e, _):
        for q in range(HH // 16):
            rows0_v[e, pl.ds(q * 16, 16)] = zero16
        return 0
    lax.fori_loop(0, CHUNK, _zrows, 0)
    for q in range(AT // CHUNK):
        pltpu.sync_copy(rows0_v, acc_s.at[pl.ds(s * AT + q * CHUNK, CHUNK), :])
    pltpu.sync_copy(rows0_v.at[pl.ds(0, AT % CHUNK), :],
                    acc_s.at[pl.ds(s * AT + (AT // CHUNK) * CHUNK,
                                   AT % CHUNK), :])
    plsc.subcore_barrier()

    G1 = (g1a_v, g1b_v)
    G2 = (g2a_v, g2b_v)
    GS1 = (gsem0, gsem1)
    GS2 = (gsem2, ssem0)

    def p1_start(q, x):
        sup = pl.ds(q * SUP, SUP)
        pltpu.async_copy(s1_s.at[src_v.at[sup]], G1[x], GS1[x])
        pltpu.async_copy(s2_s.at[dst_v.at[sup]], G2[x], GS2[x])

    def p1_wait(q, x):
        sup = pl.ds(q * SUP, SUP)
        pltpu.make_async_copy(s1_s.at[src_v.at[sup]], G1[x], GS1[x]).wait()
        pltpu.make_async_copy(s2_s.at[dst_v.at[sup]], G2[x], GS2[x]).wait()

    def p1_compute(q, x):
        def _ee(r, _):
            sl = pl.ds(r * 16, 16)
            e = G1[x][sl] + G2[x][sl]
            e = jnp.where(e > 0, e, e * ALPHA)
            att_v[pl.ds(q * SUP + r * 16, 16)] = jnp.exp(e)
            return 0
        lax.fori_loop(0, SUP // 16, _ee, 0)
        sup = pl.ds(q * SUP, SUP)
        pltpu.async_copy(att_v.at[sup], den_s.at[src_v.at[sup]], ssem1,
                         add=True)

    p1_start(0, 0)

    def _p1pair(i, _):
        q0 = i * 2
        q1 = q0 + 1
        p1_wait(q0, 0)
        p1_start(q1, 1)
        p1_compute(q0, 0)
        p1_wait(q1, 1)

        @pl.when(q1 + 1 < NSUP)
        def _():
            p1_start(q1 + 1, 0)
        p1_compute(q1, 1)
        return 0
    lax.fori_loop(0, NSUP // 2, _p1pair, 0)
    p1_wait(NSUP - 1, 0)
    p1_compute(NSUP - 1, 0)

    def _dr(q, _):
        sup = pl.ds(q * SUP, SUP)
        pltpu.make_async_copy(att_v.at[sup], den_s.at[src_v.at[sup]],
                              ssem1).wait()
        return 0
    lax.fori_loop(0, NSUP, _dr, 0)
    plsc.subcore_barrier()

    BUFS = (rows0_v, rows1_v, rows2_v)
    GS = (gsem0, gsem1, gsem2)
    SS = (ssem0, ssem1, ssem2)

    def _row_phase(wh_h, col0):
        def g_start(k, t):
            pltpu.async_copy(wh_h.at[dst_v.at[pl.ds(k * CHUNK, CHUNK)]],
                             BUFS[t], GS[t])

        def g_wait(k, t):
            pltpu.make_async_copy(
                wh_h.at[dst_v.at[pl.ds(k * CHUNK, CHUNK)]],
                BUFS[t], GS[t]).wait()

        def s_start(k, t):
            pltpu.async_copy(BUFS[t],
                             acc_s.at[src_v.at[pl.ds(k * CHUNK, CHUNK)]],
                             SS[t], add=True)

        def s_wait(k, t):
            pltpu.make_async_copy(
                BUFS[t], acc_s.at[src_v.at[pl.ds(k * CHUNK, CHUNK)]],
                SS[t]).wait()

        def _scale(k, rv):
            def _grp(g, _):
                av = att_v[pl.ds(k * CHUNK + g * 16, 16)]
                for j in range(16):
                    a_s = av[j]
                    e = g * 16 + j
                    for q in range(HH // 16):
                        sl = pl.ds(q * 16, 16)
                        rv[e, sl] = rv[e, sl] * a_s
                return 0
            lax.fori_loop(0, CHUNK // 16, _grp, 0)
            av = att_v[pl.ds(k * CHUNK + (CHUNK // 16) * 16, 16)]
            for j in range(CHUNK - (CHUNK // 16) * 16):
                a_s = av[j]
                e = (CHUNK // 16) * 16 + j
                for q in range(HH // 16):
                    sl = pl.ds(q * 16, 16)
                    rv[e, sl] = rv[e, sl] * a_s

        def _step(k, t):
            tp = (t + 2) % 3
            g_wait(k, t)
            _scale(k, BUFS[t])

            @pl.when(k >= 1)
            def _():
                s_wait(k - 1, tp)

            @pl.when(k + 2 < NCHUNK)
            def _():
                g_start(k + 2, tp)
            s_start(k, t)

        g_start(0, 0)
        g_start(1, 1)

        def _triple(i, _):
            k = i * 3
            _step(k, 0)
            _step(k + 1, 1)
            _step(k + 2, 2)
            return 0
        lax.fori_loop(0, NCHUNK // 3, _triple, 0)
        _step(NCHUNK - 1, (NCHUNK - 1) % 3)
        s_wait(NCHUNK - 1, (NCHUNK - 1) % 3)
        plsc.subcore_barrier()

        def _out_block(base, nrows):
            pltpu.sync_copy(den_s.at[pl.ds(base, nrows)],
                            zb_v.at[pl.ds(0, nrows)])
            def _inv(j, _):
                sl = pl.ds(j * 16, 16)
                d = zb_v[sl]
                zb_v[sl] = jnp.where(d > 0, 1.0 / d, 0.0)
                return 0
            lax.fori_loop(0, nrows // 16, _inv, 0)

            def _blk(r, _):
                rsl = pl.ds(base + r * 16, 16)
                pltpu.sync_copy(acc_s.at[rsl, :], rows0_v.at[pl.ds(0, 16), :])
                av = zb_v[pl.ds(r * 16, 16)]
                for j in range(16):
                    a_s = av[j]
                    for q in range(HH // 16):
                        sl = pl.ds(q * 16, 16)
                        rows0_v[j, sl] = rows0_v[j, sl] * a_s
                pltpu.sync_copy(rows0_v.at[pl.ds(0, 16), :],
                                out_h.at[rsl, pl.ds(col0, HH)])
                return 0
            lax.fori_loop(0, nrows // 16, _blk, 0)

        _out_block(s * OUT_PT, OUT_PT)

        @pl.when(s == NS - 1)
        def _():
            _out_block(NS * OUT_PT, N_NODES - NS * OUT_PT)

    @pl.when(c == 0)
    def _():
        _row_phase(wh0_h, 0)

    @pl.when(c == 1)
    def _():
        _row_phase(wh1_h, HH)


_sc_attn = functools.partial(
    pl.kernel,
    out_type=jax.ShapeDtypeStruct((N_NODES, HID), jnp.float32),
    mesh=plsc.VectorSubcoreMesh(core_axis_name="c", subcore_axis_name="s"),
    compiler_params=pltpu.CompilerParams(needs_layout_passes=False),
    scratch_types=[
        pltpu.VMEM((EPT,), jnp.int32),
        pltpu.VMEM((EPT,), jnp.int32),
        pltpu.VMEM((EPT + 16,), jnp.float32),
        pltpu.VMEM((SUP,), jnp.float32),
        pltpu.VMEM((SUP,), jnp.float32),
        pltpu.VMEM((SUP,), jnp.float32),
        pltpu.VMEM((SUP,), jnp.float32),
        pltpu.VMEM((CHUNK, HH), jnp.float32),
        pltpu.VMEM((CHUNK, HH), jnp.float32),
        pltpu.VMEM((CHUNK, HH), jnp.float32),
        pltpu.VMEM((NT,), jnp.float32),
        pltpu.SemaphoreType.DMA,
        pltpu.SemaphoreType.DMA,
        pltpu.SemaphoreType.DMA,
        pltpu.SemaphoreType.DMA,
        pltpu.SemaphoreType.DMA,
        pltpu.SemaphoreType.DMA,
        pltpu.VMEM_SHARED((NPAD,), jnp.float32),
        pltpu.VMEM_SHARED((APAD, HH), jnp.float32),
        pltpu.VMEM_SHARED((N_NODES,), jnp.float32),
        pltpu.VMEM_SHARED((N_NODES,), jnp.float32),
    ],
)(_sc_body)


def kernel(x, edge_index, W, a):
    a2 = a.reshape(2, HID).T
    wh0, wh1, s12 = _dense(x, W, a2)
    edge3 = edge_index.reshape(2, NS, EPT)
    return _sc_attn(wh0, wh1, s12[:, 0], s12[:, 1], edge3)
